# baseline (speedup 1.0000x reference)
import functools

import jax
import jax.numpy as jnp
from jax import lax
from jax.experimental import pallas as pl
from jax.experimental.pallas import tpu as pltpu
from jax.experimental.pallas import tpu_sc as plsc

L = 16
CHUNK = 128
EPS = 1e-5


def _rsqrt(v):
    i = lax.bitcast_convert_type(v, jnp.int32)
    i = jnp.int32(0x5F3759DF) - lax.shift_right_logical(i, 1)
    y = lax.bitcast_convert_type(i, jnp.float32)
    hv = 0.5 * v
    y = y * (1.5 - hv * y * y)
    y = y * (1.5 - hv * y * y)
    return y


@functools.lru_cache(maxsize=None)
def _build(B, S, V, H):
    info = plsc.get_sparse_core_info()
    NC, NS = info.num_cores, info.num_subcores
    NW = NC * NS
    N = B * S
    assert N % (NW * CHUNK) == 0
    rows_w = N // NW
    n_chunks = rows_w // CHUNK
    assert n_chunks % 2 == 0
    assert rows_w % S == 0
    KV = H // L
    inv_h = 1.0 / H

    mesh = plsc.VectorSubcoreMesh(core_axis_name="c", subcore_axis_name="s")

    @functools.partial(
        pl.kernel,
        mesh=mesh,
        out_type=jax.ShapeDtypeStruct((N * H,), jnp.float32),
        scratch_types=[
            [pltpu.VMEM((CHUNK,), jnp.int32) for _ in range(2)],
            [pltpu.VMEM((CHUNK, H), jnp.float32) for _ in range(2)],
            [pltpu.VMEM((CHUNK * H,), jnp.float32) for _ in range(2)],
            pltpu.VMEM((S, H), jnp.float32),
            pltpu.VMEM((H,), jnp.float32),
            pltpu.VMEM((H,), jnp.float32),
            [pltpu.SemaphoreType.DMA for _ in range(2)],
            [pltpu.SemaphoreType.DMA for _ in range(2)],
            [pltpu.SemaphoreType.DMA for _ in range(2)],
            pltpu.SemaphoreType.DMA,
        ],
        compiler_params=pltpu.CompilerParams(
            needs_layout_passes=False, use_tc_tiling_on_sc=True),
    )
    def emb(x_hbm, tok_hbm, pos_hbm, gamma_hbm, beta_hbm, out_hbm,
            idx_v, rows_v, out_v, pos_v, gamma_v, beta_v,
            isem, gsem, osem, psem):
        wid = lax.axis_index("s") * NC + lax.axis_index("c")
        base_w = wid * rows_w

        def pos_row(s, _):
            pltpu.async_copy(pos_hbm.at[s], pos_v.at[s], psem)
            return 0

        lax.fori_loop(0, S, pos_row, 0)
        pltpu.sync_copy(gamma_hbm, gamma_v)
        pltpu.sync_copy(beta_hbm, beta_v)
        pltpu.make_async_copy(
            pos_hbm.at[pl.ds(0, S)], pos_v, psem).wait()
        gs = [gamma_v[pl.ds(k * L, L)] for k in range(KV)]
        bs = [beta_v[pl.ds(k * L, L)] for k in range(KV)]

        def idx_start(g, p):
            pltpu.async_copy(
                x_hbm.at[pl.ds(base_w + g * CHUNK, CHUNK)], idx_v[p], isem[p])

        def idx_wait(g, p):
            pltpu.make_async_copy(
                x_hbm.at[pl.ds(base_w + g * CHUNK, CHUNK)], idx_v[p],
                isem[p]).wait()

        def gather_16(j, p):
            tvec = idx_v[p][pl.ds(j * L, L)]
            for c in range(L):
                pltpu.async_copy(
                    tok_hbm.at[tvec[c]], rows_v[p].at[j * L + c], gsem[p])

        def gather_wait(p):
            pltpu.make_async_copy(
                tok_hbm.at[pl.ds(0, CHUNK)], rows_v[p], gsem[p]).wait()

        def out_start(g, p):
            pltpu.async_copy(
                out_v[p], out_hbm.at[pl.ds((base_w + g * CHUNK) * H,
                                           CHUNK * H)], osem[p])

        def out_wait(g, p):
            pltpu.make_async_copy(
                out_v[p], out_hbm.at[pl.ds((base_w + g * CHUNK) * H,
                                           CHUNK * H)], osem[p]).wait()

        def row_compute(rv, ov, s0, i):
            s = lax.rem(s0 + i, S)
            off = i * H
            hs = [rv[i, pl.ds(k * L, L)]
                  + pos_v[s, pl.ds(k * L, L)] for k in range(KV)]
            tot = (hs[0] + hs[1]) + (hs[2] + hs[3])
            sq = (hs[0] * hs[0] + hs[1] * hs[1]) + \
                 (hs[2] * hs[2] + hs[3] * hs[3])
            s1 = plsc.cumsum(tot)[L - 1]
            s2 = plsc.cumsum(sq)[L - 1]
            mean = s1 * inv_h
            var = s2 * inv_h - mean * mean + EPS
            rstd = _rsqrt(var)
            for k in range(KV):
                ov[pl.ds(off + k * L, L)] = (hs[k] - mean) * rstd * gs[k] \
                    + bs[k]

        def compute(g, p, prefetch):
            rv, ov = rows_v[p], out_v[p]
            s0 = lax.rem(g * CHUNK, S)

            @plsc.parallel_loop(0, CHUNK // L, 1, unroll=1)
            def grp(j):
                if prefetch:
                    gather_16(j, 1 - p)
                for c in range(L):
                    row_compute(rv, ov, s0, j * L + c)

        idx_start(0, 0)
        idx_start(1, 1)
        idx_wait(0, 0)

        def g0_row(j, _):
            gather_16(j, 0)
            return 0

        lax.fori_loop(0, CHUNK // L, g0_row, 0)

        def step(g, p, prefetch):
            gather_wait(p)

            @pl.when(g + 2 < n_chunks)
            def _():
                idx_start(g + 2, p)

            @pl.when(g >= 2)
            def _():
                out_wait(g - 2, p)

            if prefetch:
                idx_wait(g + 1, 1 - p)
            compute(g, p, prefetch)
            out_start(g, p)

        def outer(gg, _):
            step(2 * gg, 0, True)
            step(2 * gg + 1, 1, True)
            return 0

        lax.fori_loop(0, n_chunks // 2 - 1, outer, 0)
        step(n_chunks - 2, 0, True)
        step(n_chunks - 1, 1, False)
        out_wait(n_chunks - 2, 0)
        out_wait(n_chunks - 1, 1)

    return emb


def kernel(x, tok_table, pos_table, gamma, beta):
    B, S = x.shape
    V, H = tok_table.shape
    emb = _build(B, S, V, H)
    out = emb(x.reshape(B * S), tok_table, pos_table, gamma, beta)
    return out.reshape(B, S, H)


# speedup vs baseline: 1.1029x; 1.1029x over previous
import functools

import jax
import jax.numpy as jnp
from jax import lax
from jax.experimental import pallas as pl
from jax.experimental.pallas import tpu as pltpu
from jax.experimental.pallas import tpu_sc as plsc

L = 16
CHUNK = 128
EPS = 1e-5


def _rsqrt(v):
    i = lax.bitcast_convert_type(v, jnp.int32)
    i = jnp.int32(0x5F3759DF) - lax.shift_right_logical(i, 1)
    y = lax.bitcast_convert_type(i, jnp.float32)
    hv = 0.5 * v
    y = y * (1.5 - hv * y * y)
    y = y * (1.5 - hv * y * y)
    return y


@functools.lru_cache(maxsize=None)
def _build(B, S, V, H):
    HP = 128
    info = plsc.get_sparse_core_info()
    NC, NS = info.num_cores, info.num_subcores
    NW = NC * NS
    N = B * S
    assert N % (NW * CHUNK) == 0
    rows_w = N // NW
    n_chunks = rows_w // CHUNK
    assert rows_w % S == 0
    KV = H // L
    inv_h = 1.0 / H

    mesh = plsc.VectorSubcoreMesh(core_axis_name="c", subcore_axis_name="s")

    @functools.partial(
        pl.kernel,
        mesh=mesh,
        out_type=jax.ShapeDtypeStruct((N, H), jnp.float32),
        scratch_types=[
            [pltpu.VMEM((CHUNK,), jnp.int32) for _ in range(2)],
            [pltpu.VMEM((CHUNK, HP), jnp.float32) for _ in range(2)],
            [pltpu.VMEM((CHUNK, H), jnp.float32) for _ in range(2)],
            pltpu.VMEM((S, H), jnp.float32),
            pltpu.VMEM((H,), jnp.float32),
            pltpu.VMEM((H,), jnp.float32),
            [pltpu.SemaphoreType.DMA for _ in range(2)],
            [pltpu.SemaphoreType.DMA for _ in range(2)],
            [pltpu.SemaphoreType.DMA for _ in range(2)],
        ],
        compiler_params=pltpu.CompilerParams(
            needs_layout_passes=False, use_tc_tiling_on_sc=True),
    )
    def emb(x_hbm, tok_hbm, pos_hbm, gamma_hbm, beta_hbm, out_hbm,
            idx_v, rows_v, out_v, pos_v, gamma_v, beta_v,
            isem, gsem, osem):
        wid = lax.axis_index("s") * NC + lax.axis_index("c")
        base_w = wid * rows_w

        pltpu.sync_copy(pos_hbm.at[pl.ds(0, S)], pos_v)
        pltpu.sync_copy(gamma_hbm, gamma_v)
        pltpu.sync_copy(beta_hbm, beta_v)
        gs = [gamma_v[pl.ds(k * L, L)] for k in range(KV)]
        bs = [beta_v[pl.ds(k * L, L)] for k in range(KV)]

        def idx_start(g, p):
            pltpu.async_copy(
                x_hbm.at[pl.ds(base_w + g * CHUNK, CHUNK)], idx_v[p], isem[p])

        def idx_wait(g, p):
            pltpu.make_async_copy(
                x_hbm.at[pl.ds(base_w + g * CHUNK, CHUNK)], idx_v[p],
                isem[p]).wait()

        def gather_start(p):
            pltpu.async_copy(tok_hbm.at[idx_v[p]], rows_v[p], gsem[p])

        def gather_wait(p):
            pltpu.make_async_copy(
                tok_hbm.at[idx_v[p]], rows_v[p], gsem[p]).wait()

        def out_start(g, p):
            pltpu.async_copy(
                out_v[p], out_hbm.at[pl.ds(base_w + g * CHUNK, CHUNK)],
                osem[p])

        def out_wait(g, p):
            pltpu.make_async_copy(
                out_v[p], out_hbm.at[pl.ds(base_w + g * CHUNK, CHUNK)],
                osem[p]).wait()

        def compute(g, p):
            rv, ov = rows_v[p], out_v[p]
            s0 = lax.rem(g * CHUNK, S)

            @plsc.parallel_loop(0, CHUNK, 1, unroll=4)
            def row(i):
                s = lax.rem(s0 + i, S)
                hs = [rv[i, pl.ds(k * L, L)] + pos_v[s, pl.ds(k * L, L)]
                      for k in range(KV)]
                tot = (hs[0] + hs[1]) + (hs[2] + hs[3])
                sq = (hs[0] * hs[0] + hs[1] * hs[1]) + \
                     (hs[2] * hs[2] + hs[3] * hs[3])
                s1 = plsc.cumsum(tot)[L - 1]
                s2 = plsc.cumsum(sq)[L - 1]
                mean = s1 * inv_h
                var = s2 * inv_h - mean * mean + EPS
                rstd = _rsqrt(var)
                for k in range(KV):
                    ov[i, pl.ds(k * L, L)] = (hs[k] - mean) * rstd * gs[k] \
                        + bs[k]

        idx_start(0, 0)
        idx_start(1, 1)
        idx_wait(0, 0)
        gather_start(0)

        def step(g, p):
            gather_wait(p)

            @pl.when(g + 2 < n_chunks)
            def _():
                idx_start(g + 2, p)

            @pl.when(g >= 2)
            def _():
                out_wait(g - 2, p)

            compute(g, p)
            out_start(g, p)

            @pl.when(g + 1 < n_chunks)
            def _():
                idx_wait(g + 1, 1 - p)
                gather_start(1 - p)

        def outer(gg, _):
            step(2 * gg, 0)
            step(2 * gg + 1, 1)
            return 0

        lax.fori_loop(0, n_chunks // 2, outer, 0)
        out_wait(n_chunks - 2, 0)
        out_wait(n_chunks - 1, 1)

    return emb


def kernel(x, tok_table, pos_table, gamma, beta):
    B, S = x.shape
    V, H = tok_table.shape
    emb = _build(B, S, V, H)
    tok_pad = jnp.pad(tok_table, ((0, 0), (0, 128 - H)))
    out = emb(x.reshape(B * S), tok_pad, pos_table, gamma, beta)
    return out.reshape(B, S, H)


# speedup vs baseline: 1.4919x; 1.3527x over previous
import functools

import jax
import jax.numpy as jnp
from jax import lax
from jax.experimental import pallas as pl
from jax.experimental.pallas import tpu as pltpu
from jax.experimental.pallas import tpu_sc as plsc

L = 16
CHUNK = 128
EPS = 1e-5


def _rsqrt(v):
    i = lax.bitcast_convert_type(v, jnp.int32)
    i = jnp.int32(0x5F3759DF) - lax.shift_right_logical(i, 1)
    y = lax.bitcast_convert_type(i, jnp.float32)
    hv = 0.5 * v
    y = y * (1.5 - hv * y * y)
    y = y * (1.5 - hv * y * y)
    return y


@functools.lru_cache(maxsize=None)
def _build(B, S, V, H):
    HP = 128
    info = plsc.get_sparse_core_info()
    NC, NS = info.num_cores, info.num_subcores
    NW = NC * NS
    N = B * S
    assert N % (NW * CHUNK) == 0
    rows_w = N // NW
    n_chunks = rows_w // CHUNK
    assert rows_w % S == 0
    KV = H // L
    inv_h = 1.0 / H

    mesh = plsc.VectorSubcoreMesh(core_axis_name="c", subcore_axis_name="s")

    @functools.partial(
        pl.kernel,
        mesh=mesh,
        out_type=jax.ShapeDtypeStruct((N, H), jnp.float32),
        scratch_types=[
            [pltpu.VMEM((CHUNK,), jnp.int32) for _ in range(2)],
            [pltpu.VMEM((CHUNK, HP), jnp.float32) for _ in range(2)],
            [pltpu.VMEM((CHUNK, H), jnp.float32) for _ in range(2)],
            pltpu.VMEM((S, H), jnp.float32),
            pltpu.VMEM((H,), jnp.float32),
            pltpu.VMEM((H,), jnp.float32),
            [pltpu.SemaphoreType.DMA for _ in range(2)],
            [pltpu.SemaphoreType.DMA for _ in range(2)],
            [pltpu.SemaphoreType.DMA for _ in range(2)],
        ],
        compiler_params=pltpu.CompilerParams(
            needs_layout_passes=False, use_tc_tiling_on_sc=True),
    )
    def emb(x_hbm, tok_hbm, pos_hbm, gamma_hbm, beta_hbm, out_hbm,
            idx_v, rows_v, out_v, pos_v, gamma_v, beta_v,
            isem, gsem, osem):
        wid = lax.axis_index("s") * NC + lax.axis_index("c")
        base_w = wid * rows_w

        pltpu.sync_copy(pos_hbm.at[pl.ds(0, S)], pos_v)
        pltpu.sync_copy(gamma_hbm, gamma_v)
        pltpu.sync_copy(beta_hbm, beta_v)
        gs = [gamma_v[pl.ds(k * L, L)] for k in range(KV)]
        bs = [beta_v[pl.ds(k * L, L)] for k in range(KV)]

        def idx_start(g, p):
            pltpu.async_copy(
                x_hbm.at[pl.ds(base_w + g * CHUNK, CHUNK)], idx_v[p], isem[p])

        def idx_wait(g, p):
            pltpu.make_async_copy(
                x_hbm.at[pl.ds(base_w + g * CHUNK, CHUNK)], idx_v[p],
                isem[p]).wait()

        def gather_start(p):
            pltpu.async_copy(tok_hbm.at[idx_v[p]], rows_v[p], gsem[p])

        def gather_wait(p):
            pltpu.make_async_copy(
                tok_hbm.at[idx_v[p]], rows_v[p], gsem[p]).wait()

        def out_start(g, p):
            pltpu.async_copy(
                out_v[p], out_hbm.at[pl.ds(base_w + g * CHUNK, CHUNK)],
                osem[p])

        def out_wait(g, p):
            pltpu.make_async_copy(
                out_v[p], out_hbm.at[pl.ds(base_w + g * CHUNK, CHUNK)],
                osem[p]).wait()

        def compute(g, p):
            rv, ov = rows_v[p], out_v[p]
            s0 = lax.rem(g * CHUNK, S)

            @plsc.parallel_loop(0, CHUNK, 1, unroll=8)
            def row(i):
                s = lax.rem(s0 + i, S)
                hs = [rv[i, pl.ds(k * L, L)] + pos_v[s, pl.ds(k * L, L)]
                      for k in range(KV)]
                tot = (hs[0] + hs[1]) + (hs[2] + hs[3])
                sq = (hs[0] * hs[0] + hs[1] * hs[1]) + \
                     (hs[2] * hs[2] + hs[3] * hs[3])
                s1 = plsc.cumsum(tot)[L - 1]
                s2 = plsc.cumsum(sq)[L - 1]
                mean = s1 * inv_h
                var = s2 * inv_h - mean * mean + EPS
                rstd = _rsqrt(var)
                for k in range(KV):
                    ov[i, pl.ds(k * L, L)] = (hs[k] - mean) * rstd * gs[k] \
                        + bs[k]

        idx_start(0, 0)
        idx_start(1, 1)
        idx_wait(0, 0)
        gather_start(0)

        def step(g, p):
            gather_wait(p)

            @pl.when(g + 2 < n_chunks)
            def _():
                idx_start(g + 2, p)

            @pl.when(g >= 2)
            def _():
                out_wait(g - 2, p)

            compute(g, p)
            out_start(g, p)

            @pl.when(g + 1 < n_chunks)
            def _():
                idx_wait(g + 1, 1 - p)
                gather_start(1 - p)

        def outer(gg, _):
            step(2 * gg, 0)
            step(2 * gg + 1, 1)
            return 0

        lax.fori_loop(0, n_chunks // 2, outer, 0)
        out_wait(n_chunks - 2, 0)
        out_wait(n_chunks - 1, 1)

    return emb


def kernel(x, tok_table, pos_table, gamma, beta):
    B, S = x.shape
    V, H = tok_table.shape
    emb = _build(B, S, V, H)
    tok_pad = jnp.pad(tok_table, ((0, 0), (0, 128 - H)))
    out = emb(x.reshape(B * S), tok_pad, pos_table, gamma, beta)
    return out.reshape(B, S, H)
